# Initial kernel scaffold; baseline (speedup 1.0000x reference)
import jax
import jax.numpy as jnp
from jax.experimental import pallas as pl


def kernel(pos, edge_index, atom_ind, means, stds, mul_w, bias_w):
    raise NotImplementedError("write your pallas kernel here")


# speedup vs baseline: 12.3902x; 12.3902x over previous
import functools
import math

import jax
import jax.numpy as jnp
from jax import lax
from jax.experimental import pallas as pl
from jax.experimental.pallas import tpu as pltpu
from jax.experimental.pallas import tpu_sc as plsc


def _make_sc_gather(E, n_nodes, n_edge_types, n_types):
    info = plsc.get_sparse_core_info()
    NC, NS = info.num_cores, info.num_subcores
    NW = NC * NS
    assert E % (NW * 16) == 0, E
    C = E // NW
    mesh = plsc.VectorSubcoreMesh(core_axis_name="c", subcore_axis_name="s")

    @functools.partial(
        pl.kernel,
        mesh=mesh,
        compiler_params=pltpu.CompilerParams(needs_layout_passes=False),
        out_type=[
            jax.ShapeDtypeStruct((E,), jnp.float32),
            jax.ShapeDtypeStruct((E,), jnp.float32),
            jax.ShapeDtypeStruct((E,), jnp.float32),
        ],
        scratch_types=[
            pltpu.VMEM((n_nodes,), jnp.float32),
            pltpu.VMEM((n_nodes,), jnp.float32),
            pltpu.VMEM((n_nodes,), jnp.float32),
            pltpu.VMEM((n_nodes,), jnp.int32),
            pltpu.VMEM((n_edge_types,), jnp.float32),
            pltpu.VMEM((n_edge_types,), jnp.float32),
            pltpu.VMEM((C,), jnp.int32),
            pltpu.VMEM((C,), jnp.int32),
            pltpu.VMEM((C,), jnp.float32),
            pltpu.VMEM((C,), jnp.float32),
            pltpu.VMEM((C,), jnp.float32),
        ],
    )
    def sc_gather(px_h, py_h, pz_h, atom_h, mulw_h, biasw_h, ei_h, ej_h,
                  d2_h, mul_h, bias_h,
                  px_v, py_v, pz_v, atom_v, mulw_v, biasw_v,
                  ei_v, ej_v, d2_v, mul_v, bias_v):
        wid = lax.axis_index("s") * NC + lax.axis_index("c")
        base = wid * C
        pltpu.sync_copy(px_h, px_v)
        pltpu.sync_copy(py_h, py_v)
        pltpu.sync_copy(pz_h, pz_v)
        pltpu.sync_copy(atom_h, atom_v)
        pltpu.sync_copy(mulw_h, mulw_v)
        pltpu.sync_copy(biasw_h, biasw_v)
        pltpu.sync_copy(ei_h.at[pl.ds(base, C)], ei_v)
        pltpu.sync_copy(ej_h.at[pl.ds(base, C)], ej_v)

        def body(i, carry):
            off = i * 16
            ei = ei_v[pl.ds(off, 16)]
            ej = ej_v[pl.ds(off, 16)]
            xi = plsc.load_gather(px_v, [ei])
            yi = plsc.load_gather(py_v, [ei])
            zi = plsc.load_gather(pz_v, [ei])
            xj = plsc.load_gather(px_v, [ej])
            yj = plsc.load_gather(py_v, [ej])
            zj = plsc.load_gather(pz_v, [ej])
            dx = xi - xj
            dy = yi - yj
            dz = zi - zj
            d2 = dx * dx + dy * dy + dz * dz
            ai = plsc.load_gather(atom_v, [ei])
            aj = plsc.load_gather(atom_v, [ej])
            t = ai * n_types + aj
            mul = plsc.load_gather(mulw_v, [t])
            bias = plsc.load_gather(biasw_v, [t])
            d2_v[pl.ds(off, 16)] = d2
            mul_v[pl.ds(off, 16)] = mul
            bias_v[pl.ds(off, 16)] = bias
            return carry

        lax.fori_loop(0, C // 16, body, 0)
        pltpu.sync_copy(d2_v, d2_h.at[pl.ds(base, C)])
        pltpu.sync_copy(mul_v, mul_h.at[pl.ds(base, C)])
        pltpu.sync_copy(bias_v, bias_h.at[pl.ds(base, C)])

    return sc_gather


def _tc_rbf(d2, mul, bias, means, stds, block_e):
    E = d2.shape[0]
    G = means.shape[1]
    inv_a = 1.0 / math.sqrt(2.0 * math.pi)

    def body(d2_ref, mul_ref, bias_ref, means_ref, stds_ref, out_ref, len_ref):
        length = jnp.sqrt(d2_ref[...])
        x = mul_ref[...] * length + bias_ref[...]
        std = jnp.abs(stds_ref[...]) + 1e-5
        inv = 1.0 / std
        z = (x - means_ref[...]) * inv
        out_ref[...] = jnp.exp(-0.5 * (z * z)) * (inv * inv_a)
        len_ref[...] = length

    return pl.pallas_call(
        body,
        grid=(E // block_e,),
        in_specs=[
            pl.BlockSpec((block_e, 1), lambda i: (i, 0)),
            pl.BlockSpec((block_e, 1), lambda i: (i, 0)),
            pl.BlockSpec((block_e, 1), lambda i: (i, 0)),
            pl.BlockSpec((1, G), lambda i: (0, 0)),
            pl.BlockSpec((1, G), lambda i: (0, 0)),
        ],
        out_specs=[
            pl.BlockSpec((block_e, G), lambda i: (i, 0)),
            pl.BlockSpec((block_e, 1), lambda i: (i, 0)),
        ],
        out_shape=[
            jax.ShapeDtypeStruct((E, G), jnp.float32),
            jax.ShapeDtypeStruct((E, 1), jnp.float32),
        ],
    )(d2.reshape(E, 1), mul.reshape(E, 1), bias.reshape(E, 1), means, stds)


def kernel(pos, edge_index, atom_ind, means, stds, mul_w, bias_w):
    E = edge_index.shape[1]
    n_nodes = pos.shape[0]
    n_edge_types = mul_w.shape[0]
    n_types = int(round(math.sqrt(n_edge_types)))
    sc = _make_sc_gather(E, n_nodes, n_edge_types, n_types)
    d2, mul, bias = sc(
        pos[:, 0], pos[:, 1], pos[:, 2], atom_ind,
        mul_w.reshape(-1), bias_w.reshape(-1),
        edge_index[0], edge_index[1],
    )
    out, length = _tc_rbf(d2, mul, bias, means, stds, block_e=2560)
    return out.astype(means.dtype), length


# speedup vs baseline: 52.0406x; 4.2001x over previous
import functools
import math

import jax
import jax.numpy as jnp
from jax import lax
from jax.experimental import pallas as pl
from jax.experimental.pallas import tpu as pltpu
from jax.experimental.pallas import tpu_sc as plsc


def _make_sc_gather(E, n_nodes, n_edge_types, n_types):
    info = plsc.get_sparse_core_info()
    NC, NS = info.num_cores, info.num_subcores
    NW = NC * NS
    assert E % (NW * 16) == 0, E
    C = E // NW
    mesh = plsc.VectorSubcoreMesh(core_axis_name="c", subcore_axis_name="s")

    @functools.partial(
        pl.kernel,
        mesh=mesh,
        compiler_params=pltpu.CompilerParams(needs_layout_passes=False),
        out_type=[
            jax.ShapeDtypeStruct((E,), jnp.float32),
            jax.ShapeDtypeStruct((E,), jnp.float32),
            jax.ShapeDtypeStruct((E,), jnp.float32),
        ],
        scratch_types=[
            pltpu.VMEM((n_nodes,), jnp.float32),
            pltpu.VMEM((n_nodes,), jnp.float32),
            pltpu.VMEM((n_nodes,), jnp.float32),
            pltpu.VMEM((n_nodes,), jnp.int32),
            pltpu.VMEM((n_edge_types,), jnp.float32),
            pltpu.VMEM((n_edge_types,), jnp.float32),
            pltpu.VMEM((C,), jnp.int32),
            pltpu.VMEM((C,), jnp.int32),
            pltpu.VMEM((C,), jnp.float32),
            pltpu.VMEM((C,), jnp.float32),
            pltpu.VMEM((C,), jnp.float32),
        ],
    )
    def sc_gather(px_h, py_h, pz_h, atom_h, mulw_h, biasw_h, ei_h, ej_h,
                  d2_h, mul_h, bias_h,
                  px_v, py_v, pz_v, atom_v, mulw_v, biasw_v,
                  ei_v, ej_v, d2_v, mul_v, bias_v):
        wid = lax.axis_index("s") * NC + lax.axis_index("c")
        base = wid * C
        pltpu.sync_copy(px_h, px_v)
        pltpu.sync_copy(py_h, py_v)
        pltpu.sync_copy(pz_h, pz_v)
        pltpu.sync_copy(atom_h, atom_v)
        pltpu.sync_copy(mulw_h, mulw_v)
        pltpu.sync_copy(biasw_h, biasw_v)
        pltpu.sync_copy(ei_h.at[pl.ds(base, C)], ei_v)
        pltpu.sync_copy(ej_h.at[pl.ds(base, C)], ej_v)

        def body(i, carry):
            off = i * 16
            ei = ei_v[pl.ds(off, 16)]
            ej = ej_v[pl.ds(off, 16)]
            xi = plsc.load_gather(px_v, [ei])
            yi = plsc.load_gather(py_v, [ei])
            zi = plsc.load_gather(pz_v, [ei])
            xj = plsc.load_gather(px_v, [ej])
            yj = plsc.load_gather(py_v, [ej])
            zj = plsc.load_gather(pz_v, [ej])
            dx = xi - xj
            dy = yi - yj
            dz = zi - zj
            d2 = dx * dx + dy * dy + dz * dz
            ai = plsc.load_gather(atom_v, [ei])
            aj = plsc.load_gather(atom_v, [ej])
            t = ai * n_types + aj
            mul = plsc.load_gather(mulw_v, [t])
            bias = plsc.load_gather(biasw_v, [t])
            d2_v[pl.ds(off, 16)] = d2
            mul_v[pl.ds(off, 16)] = mul
            bias_v[pl.ds(off, 16)] = bias
            return carry

        lax.fori_loop(0, C // 16, body, 0)
        pltpu.sync_copy(d2_v, d2_h.at[pl.ds(base, C)])
        pltpu.sync_copy(mul_v, mul_h.at[pl.ds(base, C)])
        pltpu.sync_copy(bias_v, bias_h.at[pl.ds(base, C)])

    return sc_gather


def _tc_rbf(d2, mul, bias, means, stds, block_e):
    E = d2.shape[0]
    G = means.shape[1]
    inv_a = 1.0 / math.sqrt(2.0 * math.pi)
    log2e = math.log2(math.e)
    rows = block_e // G

    nblk = E // block_e

    def body(d2_ref, mul_ref, bias_ref, means_ref, stds_ref, out_ref, len_ref):
        length_t = jnp.sqrt(d2_ref[0])
        x_t = mul_ref[0] * length_t + bias_ref[0]
        xT = x_t.T
        std = jnp.abs(stds_ref[...]) + 1e-5
        inv = 1.0 / std
        lc = jnp.log2(inv * inv_a)
        neg_half_log2e = -0.5 * log2e
        for r in range(rows):
            col = jax.lax.slice(xT, (0, r), (G, r + 1))
            z = (col - means_ref[...]) * inv
            out_ref[pl.ds(r * G, G), :] = jnp.exp2((z * z) * neg_half_log2e + lc)
        len_ref[0] = length_t

    return pl.pallas_call(
        body,
        grid=(nblk,),
        in_specs=[
            pl.BlockSpec((1, rows, G), lambda i: (i, 0, 0)),
            pl.BlockSpec((1, rows, G), lambda i: (i, 0, 0)),
            pl.BlockSpec((1, rows, G), lambda i: (i, 0, 0)),
            pl.BlockSpec((1, G), lambda i: (0, 0)),
            pl.BlockSpec((1, G), lambda i: (0, 0)),
        ],
        out_specs=[
            pl.BlockSpec((block_e, G), lambda i: (i, 0)),
            pl.BlockSpec((1, rows, G), lambda i: (i, 0, 0)),
        ],
        out_shape=[
            jax.ShapeDtypeStruct((E, G), jnp.float32),
            jax.ShapeDtypeStruct((nblk, rows, G), jnp.float32),
        ],
    )(d2.reshape(nblk, rows, G), mul.reshape(nblk, rows, G),
      bias.reshape(nblk, rows, G), means, stds)


def kernel(pos, edge_index, atom_ind, means, stds, mul_w, bias_w):
    E = edge_index.shape[1]
    n_nodes = pos.shape[0]
    n_edge_types = mul_w.shape[0]
    n_types = int(round(math.sqrt(n_edge_types)))
    sc = _make_sc_gather(E, n_nodes, n_edge_types, n_types)
    d2, mul, bias = sc(
        pos[:, 0], pos[:, 1], pos[:, 2], atom_ind,
        mul_w.reshape(-1), bias_w.reshape(-1),
        edge_index[0], edge_index[1],
    )
    out, length = _tc_rbf(d2, mul, bias, means, stds, block_e=2560)
    return out.astype(means.dtype), length.reshape(E, 1)


# speedup vs baseline: 66.6131x; 1.2800x over previous
import functools
import math

import jax
import jax.numpy as jnp
from jax import lax
from jax.experimental import pallas as pl
from jax.experimental.pallas import tpu as pltpu
from jax.experimental.pallas import tpu_sc as plsc


def _make_sc_gather(E, n_nodes, n_edge_types, n_types):
    info = plsc.get_sparse_core_info()
    NC, NS = info.num_cores, info.num_subcores
    NW = NC * NS
    assert E % (NW * 16) == 0, E
    C = E // NW
    mesh = plsc.VectorSubcoreMesh(core_axis_name="c", subcore_axis_name="s")

    @functools.partial(
        pl.kernel,
        mesh=mesh,
        compiler_params=pltpu.CompilerParams(needs_layout_passes=False),
        out_type=[
            jax.ShapeDtypeStruct((E,), jnp.float32),
            jax.ShapeDtypeStruct((E,), jnp.float32),
            jax.ShapeDtypeStruct((E,), jnp.float32),
        ],
        scratch_types=[
            pltpu.VMEM((n_nodes,), jnp.float32),
            pltpu.VMEM((n_nodes,), jnp.float32),
            pltpu.VMEM((n_nodes,), jnp.float32),
            pltpu.VMEM((n_nodes,), jnp.int32),
            pltpu.VMEM((n_edge_types,), jnp.float32),
            pltpu.VMEM((n_edge_types,), jnp.float32),
            pltpu.VMEM((C,), jnp.int32),
            pltpu.VMEM((C,), jnp.int32),
            pltpu.VMEM((C,), jnp.float32),
            pltpu.VMEM((C,), jnp.float32),
            pltpu.VMEM((C,), jnp.float32),
        ],
    )
    def sc_gather(px_h, py_h, pz_h, atom_h, mulw_h, biasw_h, ei_h, ej_h,
                  d2_h, mul_h, bias_h,
                  px_v, py_v, pz_v, atom_v, mulw_v, biasw_v,
                  ei_v, ej_v, d2_v, mul_v, bias_v):
        wid = lax.axis_index("s") * NC + lax.axis_index("c")
        base = wid * C
        pltpu.sync_copy(px_h, px_v)
        pltpu.sync_copy(py_h, py_v)
        pltpu.sync_copy(pz_h, pz_v)
        pltpu.sync_copy(atom_h, atom_v)
        pltpu.sync_copy(mulw_h, mulw_v)
        pltpu.sync_copy(biasw_h, biasw_v)
        pltpu.sync_copy(ei_h.at[pl.ds(base, C)], ei_v)
        pltpu.sync_copy(ej_h.at[pl.ds(base, C)], ej_v)

        def body(i, carry):
            off = i * 16
            ei = ei_v[pl.ds(off, 16)]
            ej = ej_v[pl.ds(off, 16)]
            xi = plsc.load_gather(px_v, [ei])
            yi = plsc.load_gather(py_v, [ei])
            zi = plsc.load_gather(pz_v, [ei])
            xj = plsc.load_gather(px_v, [ej])
            yj = plsc.load_gather(py_v, [ej])
            zj = plsc.load_gather(pz_v, [ej])
            dx = xi - xj
            dy = yi - yj
            dz = zi - zj
            d2 = dx * dx + dy * dy + dz * dz
            ai = plsc.load_gather(atom_v, [ei])
            aj = plsc.load_gather(atom_v, [ej])
            t = ai * n_types + aj
            mul = plsc.load_gather(mulw_v, [t])
            bias = plsc.load_gather(biasw_v, [t])
            d2_v[pl.ds(off, 16)] = d2
            mul_v[pl.ds(off, 16)] = mul
            bias_v[pl.ds(off, 16)] = bias
            return carry

        lax.fori_loop(0, C // 16, body, 0)
        pltpu.sync_copy(d2_v, d2_h.at[pl.ds(base, C)])
        pltpu.sync_copy(mul_v, mul_h.at[pl.ds(base, C)])
        pltpu.sync_copy(bias_v, bias_h.at[pl.ds(base, C)])

    return sc_gather


def _tc_rbf(d2, mul, bias, means, stds, block_e):
    E = d2.shape[0]
    G = means.shape[1]
    inv_a = 1.0 / math.sqrt(2.0 * math.pi)
    log2e = math.log2(math.e)
    rows = block_e // G

    nblk = E // block_e

    def body(d2_ref, mul_ref, bias_ref, means_ref, stds_ref, out_ref, len_ref):
        length_t = jnp.sqrt(d2_ref[0])
        x_t = mul_ref[0] * length_t + bias_ref[0]
        xT = x_t.T
        std = jnp.abs(stds_ref[...]) + 1e-5
        inv = 1.0 / std
        lc = jnp.log2(inv * inv_a)
        neg_half_log2e = -0.5 * log2e
        for r in range(rows):
            col = jax.lax.slice(xT, (0, r), (G, r + 1))
            z = (col - means_ref[...]) * inv
            out_ref[pl.ds(r * G, G), :] = jnp.exp2((z * z) * neg_half_log2e + lc)
        len_ref[0] = length_t

    return pl.pallas_call(
        body,
        grid=(nblk,),
        in_specs=[
            pl.BlockSpec((1, rows, G), lambda i: (i, 0, 0)),
            pl.BlockSpec((1, rows, G), lambda i: (i, 0, 0)),
            pl.BlockSpec((1, rows, G), lambda i: (i, 0, 0)),
            pl.BlockSpec((1, G), lambda i: (0, 0)),
            pl.BlockSpec((1, G), lambda i: (0, 0)),
        ],
        out_specs=[
            pl.BlockSpec((block_e, G), lambda i: (i, 0)),
            pl.BlockSpec((1, rows, G), lambda i: (i, 0, 0)),
        ],
        out_shape=[
            jax.ShapeDtypeStruct((E, G), jnp.float32),
            jax.ShapeDtypeStruct((nblk, rows, G), jnp.float32),
        ],
    )(d2.reshape(nblk, rows, G), mul.reshape(nblk, rows, G),
      bias.reshape(nblk, rows, G), means, stds)


def kernel(pos, edge_index, atom_ind, means, stds, mul_w, bias_w):
    E = edge_index.shape[1]
    n_nodes = pos.shape[0]
    n_edge_types = mul_w.shape[0]
    n_types = int(round(math.sqrt(n_edge_types)))
    sc = _make_sc_gather(E, n_nodes, n_edge_types, n_types)
    d2, mul, bias = sc(
        pos[:, 0], pos[:, 1], pos[:, 2], atom_ind,
        mul_w.reshape(-1), bias_w.reshape(-1),
        edge_index[0], edge_index[1],
    )
    out, length = _tc_rbf(d2, mul, bias, means, stds, block_e=6400)
    return out.astype(means.dtype), length.reshape(E, 1)


# speedup vs baseline: 72.5428x; 1.0890x over previous
import functools
import math

import jax
import jax.numpy as jnp
from jax import lax
from jax.experimental import pallas as pl
from jax.experimental.pallas import tpu as pltpu
from jax.experimental.pallas import tpu_sc as plsc


def _make_sc_gather(E, n_nodes, n_edge_types, n_types):
    info = plsc.get_sparse_core_info()
    NC, NS = info.num_cores, info.num_subcores
    NW = NC * NS
    assert E % (NW * 16) == 0, E
    C = E // NW
    mesh = plsc.VectorSubcoreMesh(core_axis_name="c", subcore_axis_name="s")

    @functools.partial(
        pl.kernel,
        mesh=mesh,
        compiler_params=pltpu.CompilerParams(needs_layout_passes=False),
        out_type=[
            jax.ShapeDtypeStruct((E,), jnp.float32),
            jax.ShapeDtypeStruct((E,), jnp.float32),
            jax.ShapeDtypeStruct((E,), jnp.float32),
        ],
        scratch_types=[
            pltpu.VMEM((n_nodes,), jnp.float32),
            pltpu.VMEM((n_nodes,), jnp.float32),
            pltpu.VMEM((n_nodes,), jnp.float32),
            pltpu.VMEM((n_nodes,), jnp.int32),
            pltpu.VMEM((n_edge_types,), jnp.float32),
            pltpu.VMEM((n_edge_types,), jnp.float32),
            pltpu.VMEM((C,), jnp.int32),
            pltpu.VMEM((C,), jnp.int32),
            pltpu.VMEM((C,), jnp.float32),
            pltpu.VMEM((C,), jnp.float32),
            pltpu.VMEM((C,), jnp.float32),
        ],
    )
    def sc_gather(px_h, py_h, pz_h, atom_h, mulw_h, biasw_h, ei_h, ej_h,
                  d2_h, mul_h, bias_h,
                  px_v, py_v, pz_v, atom_v, mulw_v, biasw_v,
                  ei_v, ej_v, d2_v, mul_v, bias_v):
        wid = lax.axis_index("s") * NC + lax.axis_index("c")
        base = wid * C
        pltpu.sync_copy(px_h, px_v)
        pltpu.sync_copy(py_h, py_v)
        pltpu.sync_copy(pz_h, pz_v)
        pltpu.sync_copy(atom_h, atom_v)
        pltpu.sync_copy(mulw_h, mulw_v)
        pltpu.sync_copy(biasw_h, biasw_v)
        pltpu.sync_copy(ei_h.at[pl.ds(base, C)], ei_v)
        pltpu.sync_copy(ej_h.at[pl.ds(base, C)], ej_v)

        def body(i, carry):
            off = i * 16
            ei = ei_v[pl.ds(off, 16)]
            ej = ej_v[pl.ds(off, 16)]
            xi = plsc.load_gather(px_v, [ei])
            yi = plsc.load_gather(py_v, [ei])
            zi = plsc.load_gather(pz_v, [ei])
            xj = plsc.load_gather(px_v, [ej])
            yj = plsc.load_gather(py_v, [ej])
            zj = plsc.load_gather(pz_v, [ej])
            dx = xi - xj
            dy = yi - yj
            dz = zi - zj
            d2 = dx * dx + dy * dy + dz * dz
            ai = plsc.load_gather(atom_v, [ei])
            aj = plsc.load_gather(atom_v, [ej])
            t = ai * n_types + aj
            mul = plsc.load_gather(mulw_v, [t])
            bias = plsc.load_gather(biasw_v, [t])
            d2_v[pl.ds(off, 16)] = d2
            mul_v[pl.ds(off, 16)] = mul
            bias_v[pl.ds(off, 16)] = bias
            return carry

        lax.fori_loop(0, C // 16, body, 0)
        pltpu.sync_copy(d2_v, d2_h.at[pl.ds(base, C)])
        pltpu.sync_copy(mul_v, mul_h.at[pl.ds(base, C)])
        pltpu.sync_copy(bias_v, bias_h.at[pl.ds(base, C)])

    return sc_gather


def _tc_rbf(d2, mul, bias, means, stds, block_e):
    E = d2.shape[0]
    G = means.shape[1]
    inv_a = 1.0 / math.sqrt(2.0 * math.pi)
    log2e = math.log2(math.e)
    rows = block_e // G

    nblk = E // block_e

    def body(d2_ref, mul_ref, bias_ref, means_ref, stds_ref, out_ref, len_ref):
        length_t = jnp.sqrt(d2_ref[0])
        x_t = mul_ref[0] * length_t + bias_ref[0]
        xT = x_t.T
        std = jnp.abs(stds_ref[...]) + 1e-5
        inv = 1.0 / std
        lc = jnp.log2(inv * inv_a)
        neg_half_log2e = -0.5 * log2e
        for r in range(rows):
            col = jax.lax.slice(xT, (0, r), (G, r + 1))
            z = (col - means_ref[...]) * inv
            out_ref[pl.ds(r * G, G), :] = jnp.exp2((z * z) * neg_half_log2e + lc)
        len_ref[0] = length_t

    return pl.pallas_call(
        body,
        grid=(nblk,),
        in_specs=[
            pl.BlockSpec((1, rows, G), lambda i: (i, 0, 0)),
            pl.BlockSpec((1, rows, G), lambda i: (i, 0, 0)),
            pl.BlockSpec((1, rows, G), lambda i: (i, 0, 0)),
            pl.BlockSpec((1, G), lambda i: (0, 0)),
            pl.BlockSpec((1, G), lambda i: (0, 0)),
        ],
        out_specs=[
            pl.BlockSpec((block_e, G), lambda i: (i, 0)),
            pl.BlockSpec((1, rows, G), lambda i: (i, 0, 0)),
        ],
        out_shape=[
            jax.ShapeDtypeStruct((E, G), jnp.float32),
            jax.ShapeDtypeStruct((nblk, rows, G), jnp.float32),
        ],
    )(d2.reshape(nblk, rows, G), mul.reshape(nblk, rows, G),
      bias.reshape(nblk, rows, G), means, stds)


def kernel(pos, edge_index, atom_ind, means, stds, mul_w, bias_w):
    E = edge_index.shape[1]
    n_nodes = pos.shape[0]
    n_edge_types = mul_w.shape[0]
    n_types = int(round(math.sqrt(n_edge_types)))
    sc = _make_sc_gather(E, n_nodes, n_edge_types, n_types)
    d2, mul, bias = sc(
        pos[:, 0], pos[:, 1], pos[:, 2], atom_ind,
        mul_w.reshape(-1), bias_w.reshape(-1),
        edge_index[0], edge_index[1],
    )
    out, length = _tc_rbf(d2, mul, bias, means, stds, block_e=12800)
    return out.astype(means.dtype), length.reshape(E, 1)


# speedup vs baseline: 73.6301x; 1.0150x over previous
import functools
import math

import jax
import jax.numpy as jnp
from jax import lax
from jax.experimental import pallas as pl
from jax.experimental.pallas import tpu as pltpu
from jax.experimental.pallas import tpu_sc as plsc


def _make_sc_gather(E, n_nodes, n_edge_types, n_types):
    info = plsc.get_sparse_core_info()
    NC, NS = info.num_cores, info.num_subcores
    NW = NC * NS
    assert E % (NW * 16) == 0, E
    C = E // NW
    mesh = plsc.VectorSubcoreMesh(core_axis_name="c", subcore_axis_name="s")

    @functools.partial(
        pl.kernel,
        mesh=mesh,
        compiler_params=pltpu.CompilerParams(needs_layout_passes=False),
        out_type=[
            jax.ShapeDtypeStruct((E,), jnp.float32),
            jax.ShapeDtypeStruct((E,), jnp.float32),
            jax.ShapeDtypeStruct((E,), jnp.float32),
        ],
        scratch_types=[
            pltpu.VMEM((n_nodes,), jnp.float32),
            pltpu.VMEM((n_nodes,), jnp.float32),
            pltpu.VMEM((n_nodes,), jnp.float32),
            pltpu.VMEM((n_nodes,), jnp.int32),
            pltpu.VMEM((n_edge_types,), jnp.float32),
            pltpu.VMEM((n_edge_types,), jnp.float32),
            pltpu.VMEM((C,), jnp.int32),
            pltpu.VMEM((C,), jnp.int32),
            pltpu.VMEM((C,), jnp.float32),
            pltpu.VMEM((C,), jnp.float32),
            pltpu.VMEM((C,), jnp.float32),
        ],
    )
    def sc_gather(px_h, py_h, pz_h, atom_h, mulw_h, biasw_h, ei_h, ej_h,
                  d2_h, mul_h, bias_h,
                  px_v, py_v, pz_v, atom_v, mulw_v, biasw_v,
                  ei_v, ej_v, d2_v, mul_v, bias_v):
        wid = lax.axis_index("s") * NC + lax.axis_index("c")
        base = wid * C
        pltpu.sync_copy(px_h, px_v)
        pltpu.sync_copy(py_h, py_v)
        pltpu.sync_copy(pz_h, pz_v)
        pltpu.sync_copy(atom_h, atom_v)
        pltpu.sync_copy(mulw_h, mulw_v)
        pltpu.sync_copy(biasw_h, biasw_v)
        pltpu.sync_copy(ei_h.at[pl.ds(base, C)], ei_v)
        pltpu.sync_copy(ej_h.at[pl.ds(base, C)], ej_v)

        def body(i, carry):
            off = i * 16
            ei = ei_v[pl.ds(off, 16)]
            ej = ej_v[pl.ds(off, 16)]
            xi = plsc.load_gather(px_v, [ei])
            yi = plsc.load_gather(py_v, [ei])
            zi = plsc.load_gather(pz_v, [ei])
            xj = plsc.load_gather(px_v, [ej])
            yj = plsc.load_gather(py_v, [ej])
            zj = plsc.load_gather(pz_v, [ej])
            dx = xi - xj
            dy = yi - yj
            dz = zi - zj
            d2 = dx * dx + dy * dy + dz * dz
            ai = plsc.load_gather(atom_v, [ei])
            aj = plsc.load_gather(atom_v, [ej])
            t = ai * n_types + aj
            mul = plsc.load_gather(mulw_v, [t])
            bias = plsc.load_gather(biasw_v, [t])
            d2_v[pl.ds(off, 16)] = d2
            mul_v[pl.ds(off, 16)] = mul
            bias_v[pl.ds(off, 16)] = bias
            return carry

        lax.fori_loop(0, C // 16, body, 0)
        pltpu.sync_copy(d2_v, d2_h.at[pl.ds(base, C)])
        pltpu.sync_copy(mul_v, mul_h.at[pl.ds(base, C)])
        pltpu.sync_copy(bias_v, bias_h.at[pl.ds(base, C)])

    return sc_gather


def _tc_rbf(d2, mul, bias, means, stds, block_e):
    E = d2.shape[0]
    G = means.shape[1]
    inv_a = 1.0 / math.sqrt(2.0 * math.pi)
    log2e = math.log2(math.e)
    rows = block_e // G

    nblk = E // block_e

    def body(d2_ref, mul_ref, bias_ref, means_ref, stds_ref, out_ref, len_ref):
        length_t = jnp.sqrt(d2_ref[0])
        x_t = mul_ref[0] * length_t + bias_ref[0]
        xT = x_t.T
        std = jnp.abs(stds_ref[...]) + 1e-5
        inv = 1.0 / std
        lc = jnp.log2(inv * inv_a)
        neg_half_log2e = -0.5 * log2e
        for r in range(rows):
            col = jax.lax.slice(xT, (0, r), (G, r + 1))
            z = (col - means_ref[...]) * inv
            out_ref[pl.ds(r * G, G), :] = jnp.exp2((z * z) * neg_half_log2e + lc)
        len_ref[0] = length_t

    return pl.pallas_call(
        body,
        grid=(nblk,),
        in_specs=[
            pl.BlockSpec((1, rows, G), lambda i: (i, 0, 0)),
            pl.BlockSpec((1, rows, G), lambda i: (i, 0, 0)),
            pl.BlockSpec((1, rows, G), lambda i: (i, 0, 0)),
            pl.BlockSpec((1, G), lambda i: (0, 0)),
            pl.BlockSpec((1, G), lambda i: (0, 0)),
        ],
        out_specs=[
            pl.BlockSpec((block_e, G), lambda i: (i, 0)),
            pl.BlockSpec((1, rows, G), lambda i: (i, 0, 0)),
        ],
        out_shape=[
            jax.ShapeDtypeStruct((E, G), jnp.float32),
            jax.ShapeDtypeStruct((nblk, rows, G), jnp.float32),
        ],
    )(d2.reshape(nblk, rows, G), mul.reshape(nblk, rows, G),
      bias.reshape(nblk, rows, G), means, stds)


def kernel(pos, edge_index, atom_ind, means, stds, mul_w, bias_w):
    E = edge_index.shape[1]
    n_nodes = pos.shape[0]
    n_edge_types = mul_w.shape[0]
    n_types = int(round(math.sqrt(n_edge_types)))
    sc = _make_sc_gather(E, n_nodes, n_edge_types, n_types)
    d2, mul, bias = sc(
        pos[:, 0], pos[:, 1], pos[:, 2], atom_ind,
        mul_w.reshape(-1), bias_w.reshape(-1),
        edge_index[0], edge_index[1],
    )
    out, length = _tc_rbf(d2, mul, bias, means, stds, block_e=32000)
    return out.astype(means.dtype), length.reshape(E, 1)
